# Initial kernel scaffold; baseline (speedup 1.0000x reference)
import jax
import jax.numpy as jnp
from jax.experimental import pallas as pl


def kernel(x, edge_index, W1_self, W1_neigh, b1, W2_self, W2_neigh, b2):
    raise NotImplementedError("write your pallas kernel here")


# speedup vs baseline: 4.0378x; 4.0378x over previous
import functools

import jax
import jax.numpy as jnp
from jax import lax
from jax.experimental import pallas as pl
from jax.experimental.pallas import tpu as pltpu
from jax.experimental.pallas import tpu_sc as plsc

N = 10000
E = 160000
D = 256
DH = 128

NC = 2
NS = 16
BLK = 128
KB = -(-E // (NS * BLK))
EPAD = NS * KB * BLK
NROW = ((N + NS - 1) // NS) * NS
ACC_R = 10112
DEG_R = 10240
DUMMY = N + 8


def _sc_layer_body(with_deg, tabA, tabB, src_h, dst_h, z2, z1, ones_h,
                   aggA_o, aggB_o, deg_o, acc, dacc, src_v, dst_v, rows_v,
                   ones_v, sem):
  c = lax.axis_index("c")
  s = lax.axis_index("s")

  pltpu.sync_copy(src_h.at[s], src_v)
  pltpu.sync_copy(dst_h.at[s], dst_v)

  rz = ACC_R // NS
  pltpu.sync_copy(z2, acc.at[pl.ds(s * rz, rz)])
  if with_deg:
    @pl.when(c == 1)
    def _():
      dz = DEG_R // NS
      pltpu.sync_copy(z1, dacc.at[pl.ds(s * dz, dz)])
      pltpu.sync_copy(ones_h, ones_v)
  plsc.subcore_barrier()

  def edge_loop(tab, do_deg):
    def step(j, carry):
      idx = src_v.at[j]
      pltpu.async_copy(tab.at[idx], rows_v, sem).wait()
      pltpu.sync_copy(rows_v, acc.at[dst_v.at[j]], add=True)
      if do_deg:
        pltpu.sync_copy(ones_v, dacc.at[dst_v.at[j]], add=True)
      return carry
    lax.fori_loop(0, KB, step, 0)

  @pl.when(c == 0)
  def _():
    edge_loop(tabA, False)

  @pl.when(c == 1)
  def _():
    edge_loop(tabB, with_deg)

  plsc.subcore_barrier()

  r0 = s * (ACC_R // NS)
  nr = ACC_R // NS

  @pl.when(c == 0)
  def _():
    pltpu.sync_copy(acc.at[pl.ds(r0, nr)], aggA_o.at[pl.ds(r0, nr)])

  @pl.when(c == 1)
  def _():
    pltpu.sync_copy(acc.at[pl.ds(r0, nr)], aggB_o.at[pl.ds(r0, nr)])
    if with_deg:
      d0 = s * (DEG_R // NS)
      pltpu.sync_copy(dacc.at[pl.ds(d0, DEG_R // NS)],
                      deg_o.at[pl.ds(d0, DEG_R // NS)])


def _make_sc_layer(with_deg):
  mesh = plsc.VectorSubcoreMesh(core_axis_name="c", subcore_axis_name="s",
                                num_cores=NC, num_subcores=NS)
  return pl.kernel(
      functools.partial(_sc_layer_body, with_deg),
      out_type=(
          jax.ShapeDtypeStruct((ACC_R, DH), jnp.float32),
          jax.ShapeDtypeStruct((ACC_R, DH), jnp.float32),
          jax.ShapeDtypeStruct((DEG_R,), jnp.float32),
      ),
      mesh=mesh,
      scratch_types=[
          pltpu.VMEM_SHARED((ACC_R, DH), jnp.float32),
          pltpu.VMEM_SHARED((DEG_R,), jnp.float32),
          pltpu.VMEM((KB, BLK), jnp.int32),
          pltpu.VMEM((KB, BLK), jnp.int32),
          pltpu.VMEM((BLK, DH), jnp.float32),
          pltpu.VMEM((BLK,), jnp.float32),
          pltpu.SemaphoreType.DMA,
      ],
      name="sage_segment_sum" + ("_deg" if with_deg else ""),
  )


_sc_layer_deg = _make_sc_layer(True)
_sc_layer = _make_sc_layer(False)


BM = 512
GRID_M = -(-N // BM)


def _tc1_body(x_ref, w_ref, b_ref, xs_ref, xnA_ref, xnB_ref):
  y = jnp.dot(x_ref[...], w_ref[...], preferred_element_type=jnp.float32)
  xs_ref[...] = y[:, :D] + b_ref[...]
  xnA_ref[...] = y[:, D:D + DH]
  xnB_ref[...] = y[:, D + DH:]


def _tc2_body(xs_ref, aA_ref, aB_ref, d_ref, w_ref, b_ref,
              hs_ref, hnA_ref, hnB_ref):
  invd = 1.0 / jnp.maximum(d_ref[...], 1.0)
  agg = jnp.concatenate([aA_ref[...], aB_ref[...]], axis=1) * invd
  h = jnp.maximum(xs_ref[...] + agg, 0.0)
  y = jnp.dot(h, w_ref[...], preferred_element_type=jnp.float32)
  hs_ref[...] = y[:, :D] + b_ref[...]
  hnA_ref[...] = y[:, D:D + DH]
  hnB_ref[...] = y[:, D + DH:]


def _tc3_body(hs_ref, aA_ref, aB_ref, d_ref, o_ref):
  invd = 1.0 / jnp.maximum(d_ref[...], 1.0)
  agg = jnp.concatenate([aA_ref[...], aB_ref[...]], axis=1) * invd
  o_ref[...] = hs_ref[...] + agg


def _row_spec(cols):
  return pl.BlockSpec((BM, cols), lambda i: (i, 0))


_W_SPEC = pl.BlockSpec((D, 2 * D), lambda i: (0, 0))
_B_SPEC = pl.BlockSpec((1, D), lambda i: (0, 0))

_tc1 = pl.pallas_call(
    _tc1_body,
    grid=(GRID_M,),
    in_specs=[_row_spec(D), _W_SPEC, _B_SPEC],
    out_specs=[_row_spec(D), _row_spec(DH), _row_spec(DH)],
    out_shape=[
        jax.ShapeDtypeStruct((N, D), jnp.float32),
        jax.ShapeDtypeStruct((N, DH), jnp.float32),
        jax.ShapeDtypeStruct((N, DH), jnp.float32),
    ],
    compiler_params=pltpu.CompilerParams(
        dimension_semantics=("parallel",)),
)

_tc2 = pl.pallas_call(
    _tc2_body,
    grid=(GRID_M,),
    in_specs=[_row_spec(D), _row_spec(DH), _row_spec(DH),
              pl.BlockSpec((BM, 1), lambda i: (i, 0)), _W_SPEC, _B_SPEC],
    out_specs=[_row_spec(D), _row_spec(DH), _row_spec(DH)],
    out_shape=[
        jax.ShapeDtypeStruct((N, D), jnp.float32),
        jax.ShapeDtypeStruct((N, DH), jnp.float32),
        jax.ShapeDtypeStruct((N, DH), jnp.float32),
    ],
    compiler_params=pltpu.CompilerParams(
        dimension_semantics=("parallel",)),
)

_tc3 = pl.pallas_call(
    _tc3_body,
    grid=(GRID_M,),
    in_specs=[_row_spec(D), _row_spec(DH), _row_spec(DH),
              pl.BlockSpec((BM, 1), lambda i: (i, 0))],
    out_specs=_row_spec(D),
    out_shape=jax.ShapeDtypeStruct((N, D), jnp.float32),
    compiler_params=pltpu.CompilerParams(
        dimension_semantics=("parallel",)),
)


@jax.jit
def kernel(x, edge_index, W1_self, W1_neigh, b1, W2_self, W2_neigh, b2):
  W1 = jnp.concatenate([W1_self, W1_neigh], axis=1)
  W2 = jnp.concatenate([W2_self, W2_neigh], axis=1)

  src = edge_index[0]
  dst = edge_index[1]
  pad = EPAD - E
  srcp = jnp.concatenate([src, jnp.zeros((pad,), jnp.int32)]).reshape(
      NS, KB, BLK)
  dstp = jnp.concatenate([dst, jnp.full((pad,), DUMMY, jnp.int32)]).reshape(
      NS, KB, BLK)

  z2 = jnp.zeros((ACC_R // NS, DH), jnp.float32)
  z1 = jnp.zeros((DEG_R // NS,), jnp.float32)
  ones = jnp.ones((BLK,), jnp.float32)

  xs, xnA, xnB = _tc1(x, W1, b1.reshape(1, D))
  aggA, aggB, deg = _sc_layer_deg(xnA, xnB, srcp, dstp, z2, z1, ones)
  d = deg.reshape(DEG_R, 1)

  hs, hnA, hnB = _tc2(xs, aggA, aggB, d, W2, b2.reshape(1, D))
  a2A, a2B, _ = _sc_layer(hnA, hnB, srcp, dstp, z2, z1, ones)
  out = _tc3(hs, a2A, a2B, d)
  return out


# speedup vs baseline: 5.0137x; 1.2417x over previous
import functools

import jax
import jax.numpy as jnp
from jax import lax
from jax.experimental import pallas as pl
from jax.experimental.pallas import tpu as pltpu
from jax.experimental.pallas import tpu_sc as plsc

N = 10000
E = 160000
D = 256
DH = 128

NC = 2
NS = 16
BLK = 128
KB = -(-E // (NS * BLK))
EPAD = NS * KB * BLK
NROW = ((N + NS - 1) // NS) * NS
ACC_R = 10112
DEG_R = 10240
DUMMY = N + 8


def _sc_layer_body(with_deg, tabA, tabB, pk_h, z2, z1, ones_h,
                   aggA_o, aggB_o, deg_o, acc, dacc, pk_v, sidx, didx,
                   rows_v, ones_v, gsem, ssem, dsem):
  c = lax.axis_index("c")
  s = lax.axis_index("s")

  pltpu.sync_copy(pk_h.at[s], pk_v)

  rz = ACC_R // NS
  pltpu.sync_copy(z2, acc.at[pl.ds(s * rz, rz)])
  if with_deg:
    @pl.when(c == 1)
    def _():
      dz = DEG_R // NS
      pltpu.sync_copy(z1, dacc.at[pl.ds(s * dz, dz)])
      pltpu.sync_copy(ones_h, ones_v)
  plsc.subcore_barrier()

  def unpack_idx(jb, buf):
    row = pk_v.at[jb]
    for i in range(BLK // 16):
      p = row[pl.ds(i * 16, 16)]
      sidx[buf, pl.ds(i * 16, 16)] = p & 0xFFFF
      didx[buf, pl.ds(i * 16, 16)] = lax.shift_right_logical(p, 16)

  def edge_loop(tab, do_deg):
    unpack_idx(0, 0)
    pltpu.async_copy(tab.at[sidx.at[0]], rows_v.at[0], gsem)

    def step(j, carry):
      buf = lax.rem(j, 2)
      obuf = 1 - buf
      pltpu.make_async_copy(tab.at[sidx.at[buf]], rows_v.at[buf],
                            gsem).wait()
      pltpu.async_copy(rows_v.at[buf], acc.at[didx.at[buf]], ssem, add=True)
      if do_deg:
        pltpu.async_copy(ones_v, dacc.at[didx.at[buf]], dsem, add=True)

      @pl.when(j >= 1)
      def _():
        pltpu.make_async_copy(rows_v.at[obuf], acc.at[didx.at[obuf]],
                              ssem).wait()
        if do_deg:
          pltpu.make_async_copy(ones_v, dacc.at[didx.at[obuf]],
                                dsem).wait()

      @pl.when(j + 1 < KB)
      def _():
        unpack_idx(j + 1, obuf)
        pltpu.async_copy(tab.at[sidx.at[obuf]], rows_v.at[obuf], gsem)
      return carry

    lax.fori_loop(0, KB, step, 0)
    fbuf = (KB - 1) % 2
    pltpu.make_async_copy(rows_v.at[fbuf], acc.at[didx.at[fbuf]],
                          ssem).wait()
    if do_deg:
      pltpu.make_async_copy(ones_v, dacc.at[didx.at[fbuf]], dsem).wait()

  @pl.when(c == 0)
  def _():
    edge_loop(tabA, False)

  @pl.when(c == 1)
  def _():
    edge_loop(tabB, with_deg)

  plsc.subcore_barrier()

  r0 = s * (ACC_R // NS)
  nr = ACC_R // NS

  @pl.when(c == 0)
  def _():
    pltpu.sync_copy(acc.at[pl.ds(r0, nr)], aggA_o.at[pl.ds(r0, nr)])

  @pl.when(c == 1)
  def _():
    pltpu.sync_copy(acc.at[pl.ds(r0, nr)], aggB_o.at[pl.ds(r0, nr)])
    if with_deg:
      d0 = s * (DEG_R // NS)
      pltpu.sync_copy(dacc.at[pl.ds(d0, DEG_R // NS)],
                      deg_o.at[pl.ds(d0, DEG_R // NS)])


def _make_sc_layer(with_deg):
  mesh = plsc.VectorSubcoreMesh(core_axis_name="c", subcore_axis_name="s",
                                num_cores=NC, num_subcores=NS)
  return pl.kernel(
      functools.partial(_sc_layer_body, with_deg),
      out_type=(
          jax.ShapeDtypeStruct((ACC_R, DH), jnp.float32),
          jax.ShapeDtypeStruct((ACC_R, DH), jnp.float32),
          jax.ShapeDtypeStruct((DEG_R,), jnp.float32),
      ),
      mesh=mesh,
      scratch_types=[
          pltpu.VMEM_SHARED((ACC_R, DH), jnp.float32),
          pltpu.VMEM_SHARED((DEG_R,), jnp.float32),
          pltpu.VMEM((KB, BLK), jnp.int32),
          pltpu.VMEM((2, BLK), jnp.int32),
          pltpu.VMEM((2, BLK), jnp.int32),
          pltpu.VMEM((2, BLK, DH), jnp.float32),
          pltpu.VMEM((BLK,), jnp.float32),
          pltpu.SemaphoreType.DMA,
          pltpu.SemaphoreType.DMA,
          pltpu.SemaphoreType.DMA,
      ],
      name="sage_segment_sum" + ("_deg" if with_deg else ""),
  )


_sc_layer_deg = _make_sc_layer(True)
_sc_layer = _make_sc_layer(False)


BM = 512
GRID_M = -(-N // BM)


def _tc1_body(x_ref, w_ref, b_ref, xs_ref, xnA_ref, xnB_ref):
  y = jnp.dot(x_ref[...], w_ref[...], preferred_element_type=jnp.float32)
  xs_ref[...] = y[:, :D] + b_ref[...]
  xnA_ref[...] = y[:, D:D + DH]
  xnB_ref[...] = y[:, D + DH:]


def _tc2_body(xs_ref, aA_ref, aB_ref, d_ref, w_ref, b_ref,
              hs_ref, hnA_ref, hnB_ref):
  invd = 1.0 / jnp.maximum(d_ref[...], 1.0)
  agg = jnp.concatenate([aA_ref[...], aB_ref[...]], axis=1) * invd
  h = jnp.maximum(xs_ref[...] + agg, 0.0)
  y = jnp.dot(h, w_ref[...], preferred_element_type=jnp.float32)
  hs_ref[...] = y[:, :D] + b_ref[...]
  hnA_ref[...] = y[:, D:D + DH]
  hnB_ref[...] = y[:, D + DH:]


def _tc3_body(hs_ref, aA_ref, aB_ref, d_ref, o_ref):
  invd = 1.0 / jnp.maximum(d_ref[...], 1.0)
  agg = jnp.concatenate([aA_ref[...], aB_ref[...]], axis=1) * invd
  o_ref[...] = hs_ref[...] + agg


def _row_spec(cols):
  return pl.BlockSpec((BM, cols), lambda i: (i, 0))


_W_SPEC = pl.BlockSpec((D, 2 * D), lambda i: (0, 0))
_B_SPEC = pl.BlockSpec((1, D), lambda i: (0, 0))

_tc1 = pl.pallas_call(
    _tc1_body,
    grid=(GRID_M,),
    in_specs=[_row_spec(D), _W_SPEC, _B_SPEC],
    out_specs=[_row_spec(D), _row_spec(DH), _row_spec(DH)],
    out_shape=[
        jax.ShapeDtypeStruct((N, D), jnp.float32),
        jax.ShapeDtypeStruct((N, DH), jnp.float32),
        jax.ShapeDtypeStruct((N, DH), jnp.float32),
    ],
    compiler_params=pltpu.CompilerParams(
        dimension_semantics=("parallel",)),
)

_tc2 = pl.pallas_call(
    _tc2_body,
    grid=(GRID_M,),
    in_specs=[_row_spec(D), _row_spec(DH), _row_spec(DH),
              pl.BlockSpec((BM, 1), lambda i: (i, 0)), _W_SPEC, _B_SPEC],
    out_specs=[_row_spec(D), _row_spec(DH), _row_spec(DH)],
    out_shape=[
        jax.ShapeDtypeStruct((N, D), jnp.float32),
        jax.ShapeDtypeStruct((N, DH), jnp.float32),
        jax.ShapeDtypeStruct((N, DH), jnp.float32),
    ],
    compiler_params=pltpu.CompilerParams(
        dimension_semantics=("parallel",)),
)

_tc3 = pl.pallas_call(
    _tc3_body,
    grid=(GRID_M,),
    in_specs=[_row_spec(D), _row_spec(DH), _row_spec(DH),
              pl.BlockSpec((BM, 1), lambda i: (i, 0))],
    out_specs=_row_spec(D),
    out_shape=jax.ShapeDtypeStruct((N, D), jnp.float32),
    compiler_params=pltpu.CompilerParams(
        dimension_semantics=("parallel",)),
)


@jax.jit
def kernel(x, edge_index, W1_self, W1_neigh, b1, W2_self, W2_neigh, b2):
  W1 = jnp.concatenate([W1_self, W1_neigh], axis=1)
  W2 = jnp.concatenate([W2_self, W2_neigh], axis=1)

  src = edge_index[0]
  dst = edge_index[1]
  pad = EPAD - E
  packed = jnp.concatenate([
      (dst << 16) | src,
      jnp.full((pad,), DUMMY << 16, jnp.int32),
  ]).reshape(NS, KB, BLK)

  z2 = jnp.zeros((ACC_R // NS, DH), jnp.float32)
  z1 = jnp.zeros((DEG_R // NS,), jnp.float32)
  ones = jnp.ones((BLK,), jnp.float32)

  xs, xnA, xnB = _tc1(x, W1, b1.reshape(1, D))
  aggA, aggB, deg = _sc_layer_deg(xnA, xnB, packed, z2, z1, ones)
  d = deg.reshape(DEG_R, 1)

  hs, hnA, hnB = _tc2(xs, aggA, aggB, d, W2, b2.reshape(1, D))
  a2A, a2B, _ = _sc_layer(hnA, hnB, packed, z2, z1, ones)
  out = _tc3(hs, a2A, a2B, d)
  return out


# speedup vs baseline: 5.0614x; 1.0095x over previous
import functools

import jax
import jax.numpy as jnp
from jax import lax
from jax.experimental import pallas as pl
from jax.experimental.pallas import tpu as pltpu
from jax.experimental.pallas import tpu_sc as plsc

N = 10000
E = 160000
D = 256
DH = 128

NC = 2
NS = 16
BLK = 128
KB = -(-E // (NS * BLK))
EPAD = NS * KB * BLK
ACC_R = 10112
DEG_R = 10240
DUMMY = N + 8


def _sc_layer_body(with_deg, tabA, tabB, pk_h, z2, z1, ones_h,
                   aggA_o, aggB_o, deg_o, acc, dacc, pk_v, sidx, didx,
                   rows_v, ones_v, gsem, ssem, dsem):
  c = lax.axis_index("c")
  s = lax.axis_index("s")

  pltpu.sync_copy(pk_h.at[s], pk_v)

  rz = ACC_R // NS
  pltpu.sync_copy(z2, acc.at[pl.ds(s * rz, rz)])
  if with_deg:
    @pl.when(c == 1)
    def _():
      dz = DEG_R // NS
      pltpu.sync_copy(z1, dacc.at[pl.ds(s * dz, dz)])
      pltpu.sync_copy(ones_h, ones_v)
  plsc.subcore_barrier()

  def unpack_idx(jb, buf):
    row = pk_v.at[jb]
    for i in range(BLK // 16):
      p = row[pl.ds(i * 16, 16)]
      sidx[buf, pl.ds(i * 16, 16)] = p & 0xFFFF
      didx[buf, pl.ds(i * 16, 16)] = lax.shift_right_logical(p, 16)

  def edge_loop(tab, do_deg):
    unpack_idx(0, 0)
    pltpu.async_copy(tab.at[sidx.at[0]], rows_v.at[0], gsem)

    def step(j, carry):
      buf = lax.rem(j, 2)
      obuf = 1 - buf
      pltpu.make_async_copy(tab.at[sidx.at[buf]], rows_v.at[buf],
                            gsem).wait()
      pltpu.async_copy(rows_v.at[buf], acc.at[didx.at[buf]], ssem, add=True)
      if do_deg:
        pltpu.async_copy(ones_v, dacc.at[didx.at[buf]], dsem, add=True)

      @pl.when(j >= 1)
      def _():
        pltpu.make_async_copy(rows_v.at[obuf], acc.at[didx.at[obuf]],
                              ssem).wait()
        if do_deg:
          pltpu.make_async_copy(ones_v, dacc.at[didx.at[obuf]],
                                dsem).wait()

      @pl.when(j + 1 < KB)
      def _():
        unpack_idx(j + 1, obuf)
        pltpu.async_copy(tab.at[sidx.at[obuf]], rows_v.at[obuf], gsem)
      return carry

    lax.fori_loop(0, KB, step, 0)
    fbuf = (KB - 1) % 2
    pltpu.make_async_copy(rows_v.at[fbuf], acc.at[didx.at[fbuf]],
                          ssem).wait()
    if do_deg:
      pltpu.make_async_copy(ones_v, dacc.at[didx.at[fbuf]], dsem).wait()

  @pl.when(c == 0)
  def _():
    edge_loop(tabA, False)

  @pl.when(c == 1)
  def _():
    edge_loop(tabB, with_deg)

  plsc.subcore_barrier()

  r0 = s * (ACC_R // NS)
  nr = ACC_R // NS

  @pl.when(c == 0)
  def _():
    pltpu.sync_copy(acc.at[pl.ds(r0, nr)], aggA_o.at[pl.ds(r0, nr)])

  @pl.when(c == 1)
  def _():
    pltpu.sync_copy(acc.at[pl.ds(r0, nr)], aggB_o.at[pl.ds(r0, nr)])
    if with_deg:
      d0 = s * (DEG_R // NS)
      pltpu.sync_copy(dacc.at[pl.ds(d0, DEG_R // NS)],
                      deg_o.at[pl.ds(d0, DEG_R // NS)])


def _make_sc_layer(with_deg):
  mesh = plsc.VectorSubcoreMesh(core_axis_name="c", subcore_axis_name="s",
                                num_cores=NC, num_subcores=NS)
  return pl.kernel(
      functools.partial(_sc_layer_body, with_deg),
      out_type=(
          jax.ShapeDtypeStruct((ACC_R, DH), jnp.float32),
          jax.ShapeDtypeStruct((ACC_R, DH), jnp.float32),
          jax.ShapeDtypeStruct((DEG_R,), jnp.float32),
      ),
      mesh=mesh,
      scratch_types=[
          pltpu.VMEM_SHARED((ACC_R, DH), jnp.float32),
          pltpu.VMEM_SHARED((DEG_R,), jnp.float32),
          pltpu.VMEM((KB, BLK), jnp.int32),
          pltpu.VMEM((2, BLK), jnp.int32),
          pltpu.VMEM((2, BLK), jnp.int32),
          pltpu.VMEM((2, BLK, DH), jnp.float32),
          pltpu.VMEM((BLK,), jnp.float32),
          pltpu.SemaphoreType.DMA,
          pltpu.SemaphoreType.DMA,
          pltpu.SemaphoreType.DMA,
      ],
      name="sage_segment_sum" + ("_deg" if with_deg else ""),
  )


_sc_layer_deg = _make_sc_layer(True)
_sc_layer = _make_sc_layer(False)


BM = 512
GRID_M = -(-N // BM)


def _tc1_body(x_ref, w_ref, b_ref, xs_ref, xnA_ref, xnB_ref):
  y = jnp.dot(x_ref[...].astype(jnp.bfloat16), w_ref[...],
              preferred_element_type=jnp.float32)
  xs_ref[...] = y[:, :D] + b_ref[...]
  xnA_ref[...] = y[:, D:D + DH]
  xnB_ref[...] = y[:, D + DH:]


def _tc2_body(xs_ref, aA_ref, aB_ref, d_ref, w_ref, b_ref,
              hs_ref, hnA_ref, hnB_ref):
  invd = 1.0 / jnp.maximum(d_ref[...], 1.0)
  agg = jnp.concatenate([aA_ref[...], aB_ref[...]], axis=1) * invd
  h = jnp.maximum(xs_ref[...] + agg, 0.0)
  y = jnp.dot(h.astype(jnp.bfloat16), w_ref[...],
              preferred_element_type=jnp.float32)
  hs_ref[...] = y[:, :D] + b_ref[...]
  hnA_ref[...] = y[:, D:D + DH]
  hnB_ref[...] = y[:, D + DH:]


def _tc3_body(hs_ref, aA_ref, aB_ref, d_ref, o_ref):
  invd = 1.0 / jnp.maximum(d_ref[...], 1.0)
  agg = jnp.concatenate([aA_ref[...], aB_ref[...]], axis=1) * invd
  o_ref[...] = hs_ref[...] + agg


def _row_spec(cols):
  return pl.BlockSpec((BM, cols), lambda i: (i, 0))


_W_SPEC = pl.BlockSpec((D, 2 * D), lambda i: (0, 0))
_B_SPEC = pl.BlockSpec((1, D), lambda i: (0, 0))
_D_SPEC = pl.BlockSpec((BM, 1), lambda i: (i, 0))

_tc1 = pl.pallas_call(
    _tc1_body,
    grid=(GRID_M,),
    in_specs=[_row_spec(D), _W_SPEC, _B_SPEC],
    out_specs=[_row_spec(D), _row_spec(DH), _row_spec(DH)],
    out_shape=[
        jax.ShapeDtypeStruct((N, D), jnp.float32),
        jax.ShapeDtypeStruct((N, DH), jnp.float32),
        jax.ShapeDtypeStruct((N, DH), jnp.float32),
    ],
    compiler_params=pltpu.CompilerParams(
        dimension_semantics=("parallel",)),
)

_tc2 = pl.pallas_call(
    _tc2_body,
    grid=(GRID_M,),
    in_specs=[_row_spec(D), _row_spec(DH), _row_spec(DH), _D_SPEC,
              _W_SPEC, _B_SPEC],
    out_specs=[_row_spec(D), _row_spec(DH), _row_spec(DH)],
    out_shape=[
        jax.ShapeDtypeStruct((N, D), jnp.float32),
        jax.ShapeDtypeStruct((N, DH), jnp.float32),
        jax.ShapeDtypeStruct((N, DH), jnp.float32),
    ],
    compiler_params=pltpu.CompilerParams(
        dimension_semantics=("parallel",)),
)

_tc3 = pl.pallas_call(
    _tc3_body,
    grid=(GRID_M,),
    in_specs=[_row_spec(D), _row_spec(DH), _row_spec(DH), _D_SPEC],
    out_specs=_row_spec(D),
    out_shape=jax.ShapeDtypeStruct((N, D), jnp.float32),
    compiler_params=pltpu.CompilerParams(
        dimension_semantics=("parallel",)),
)


@jax.jit
def kernel(x, edge_index, W1_self, W1_neigh, b1, W2_self, W2_neigh, b2):
  W1 = jnp.concatenate([W1_self, W1_neigh], axis=1).astype(jnp.bfloat16)
  W2 = jnp.concatenate([W2_self, W2_neigh], axis=1).astype(jnp.bfloat16)

  src = edge_index[0]
  dst = edge_index[1]
  pad = EPAD - E
  packed = jnp.concatenate([
      (dst << 16) | src,
      jnp.full((pad,), DUMMY << 16, jnp.int32),
  ]).reshape(NS, KB, BLK)

  z2 = jnp.zeros((ACC_R // NS, DH), jnp.float32)
  z1 = jnp.zeros((DEG_R // NS,), jnp.float32)
  ones = jnp.ones((BLK,), jnp.float32)

  xs, xnA, xnB = _tc1(x, W1, b1.reshape(1, D))
  aggA, aggB, deg = _sc_layer_deg(xnA, xnB, packed, z2, z1, ones)
  d = deg.reshape(DEG_R, 1)

  hs, hnA, hnB = _tc2(xs, aggA, aggB, d, W2, b2.reshape(1, D))
  a2A, a2B, _ = _sc_layer(hnA, hnB, packed, z2, z1, ones)
  out = _tc3(hs, a2A, a2B, d)
  return out


# speedup vs baseline: 5.0857x; 1.0048x over previous
import functools

import jax
import jax.numpy as jnp
from jax import lax
from jax.experimental import pallas as pl
from jax.experimental.pallas import tpu as pltpu
from jax.experimental.pallas import tpu_sc as plsc

N = 10000
E = 160000
D = 256
DH = 128

NC = 2
NS = 16
BLK = 128
KB = -(-E // (NS * BLK))
EPAD = NS * KB * BLK
ACC_R = 10112
DEG_R = 10240
DUMMY = N + 8


def _sc_layer_body(with_deg, tabA, tabB, pk_h, z2, z1, ones_h,
                   aggA_o, aggB_o, deg_o, acc, dacc, pk_v, sidx, didx,
                   rows_v, ones_v, gsem, ssem, dsem):
  c = lax.axis_index("c")
  s = lax.axis_index("s")

  pltpu.sync_copy(pk_h.at[s], pk_v)

  rz = ACC_R // NS
  pltpu.sync_copy(z2, acc.at[pl.ds(s * rz, rz)])
  if with_deg:
    @pl.when(c == 1)
    def _():
      dz = DEG_R // NS
      pltpu.sync_copy(z1, dacc.at[pl.ds(s * dz, dz)])
      pltpu.sync_copy(ones_h, ones_v)
  plsc.subcore_barrier()

  def unpack_idx(jb, buf):
    row = pk_v.at[jb]
    for i in range(BLK // 16):
      p = row[pl.ds(i * 16, 16)]
      sidx[buf, pl.ds(i * 16, 16)] = p & 0xFFFF
      didx[buf, pl.ds(i * 16, 16)] = lax.shift_right_logical(p, 16)

  def edge_loop(tab, do_deg):
    H = BLK // 2

    def issue_gather(buf):
      pltpu.async_copy(tab.at[sidx.at[buf, pl.ds(0, H)]],
                       rows_v.at[buf, pl.ds(0, H)], gsem)
      pltpu.async_copy(tab.at[sidx.at[buf, pl.ds(H, H)]],
                       rows_v.at[buf, pl.ds(H, H)], gsem)

    def wait_gather(buf):
      pltpu.make_async_copy(tab.at[sidx.at[buf, pl.ds(0, H)]],
                            rows_v.at[buf, pl.ds(0, H)], gsem).wait()
      pltpu.make_async_copy(tab.at[sidx.at[buf, pl.ds(H, H)]],
                            rows_v.at[buf, pl.ds(H, H)], gsem).wait()

    unpack_idx(0, 0)
    issue_gather(0)

    def step(j, carry):
      buf = lax.rem(j, 2)
      obuf = 1 - buf
      wait_gather(buf)
      pltpu.async_copy(rows_v.at[buf], acc.at[didx.at[buf]], ssem, add=True)
      if do_deg:
        pltpu.async_copy(ones_v, dacc.at[didx.at[buf]], dsem, add=True)

      @pl.when(j >= 1)
      def _():
        pltpu.make_async_copy(rows_v.at[obuf], acc.at[didx.at[obuf]],
                              ssem).wait()
        if do_deg:
          pltpu.make_async_copy(ones_v, dacc.at[didx.at[obuf]],
                                dsem).wait()

      @pl.when(j + 1 < KB)
      def _():
        unpack_idx(j + 1, obuf)
        issue_gather(obuf)
      return carry

    lax.fori_loop(0, KB, step, 0)
    fbuf = (KB - 1) % 2
    pltpu.make_async_copy(rows_v.at[fbuf], acc.at[didx.at[fbuf]],
                          ssem).wait()
    if do_deg:
      pltpu.make_async_copy(ones_v, dacc.at[didx.at[fbuf]], dsem).wait()

  @pl.when(c == 0)
  def _():
    edge_loop(tabA, False)

  @pl.when(c == 1)
  def _():
    edge_loop(tabB, with_deg)

  plsc.subcore_barrier()

  r0 = s * (ACC_R // NS)
  nr = ACC_R // NS

  @pl.when(c == 0)
  def _():
    pltpu.sync_copy(acc.at[pl.ds(r0, nr)], aggA_o.at[pl.ds(r0, nr)])

  @pl.when(c == 1)
  def _():
    pltpu.sync_copy(acc.at[pl.ds(r0, nr)], aggB_o.at[pl.ds(r0, nr)])
    if with_deg:
      d0 = s * (DEG_R // NS)
      pltpu.sync_copy(dacc.at[pl.ds(d0, DEG_R // NS)],
                      deg_o.at[pl.ds(d0, DEG_R // NS)])


def _make_sc_layer(with_deg):
  mesh = plsc.VectorSubcoreMesh(core_axis_name="c", subcore_axis_name="s",
                                num_cores=NC, num_subcores=NS)
  return pl.kernel(
      functools.partial(_sc_layer_body, with_deg),
      out_type=(
          jax.ShapeDtypeStruct((ACC_R, DH), jnp.float32),
          jax.ShapeDtypeStruct((ACC_R, DH), jnp.float32),
          jax.ShapeDtypeStruct((DEG_R,), jnp.float32),
      ),
      mesh=mesh,
      scratch_types=[
          pltpu.VMEM_SHARED((ACC_R, DH), jnp.float32),
          pltpu.VMEM_SHARED((DEG_R,), jnp.float32),
          pltpu.VMEM((KB, BLK), jnp.int32),
          pltpu.VMEM((2, BLK), jnp.int32),
          pltpu.VMEM((2, BLK), jnp.int32),
          pltpu.VMEM((2, BLK, DH), jnp.float32),
          pltpu.VMEM((BLK,), jnp.float32),
          pltpu.SemaphoreType.DMA,
          pltpu.SemaphoreType.DMA,
          pltpu.SemaphoreType.DMA,
      ],
      name="sage_segment_sum" + ("_deg" if with_deg else ""),
  )


_sc_layer_deg = _make_sc_layer(True)
_sc_layer = _make_sc_layer(False)


BM = 512
GRID_M = -(-N // BM)


def _tc1_body(x_ref, w_ref, b_ref, xs_ref, xnA_ref, xnB_ref):
  y = jnp.dot(x_ref[...].astype(jnp.bfloat16), w_ref[...],
              preferred_element_type=jnp.float32)
  xs_ref[...] = y[:, :D] + b_ref[...]
  xnA_ref[...] = y[:, D:D + DH]
  xnB_ref[...] = y[:, D + DH:]


def _tc2_body(xs_ref, aA_ref, aB_ref, d_ref, w_ref, b_ref,
              hs_ref, hnA_ref, hnB_ref):
  invd = 1.0 / jnp.maximum(d_ref[...], 1.0)
  agg = jnp.concatenate([aA_ref[...], aB_ref[...]], axis=1) * invd
  h = jnp.maximum(xs_ref[...] + agg, 0.0)
  y = jnp.dot(h.astype(jnp.bfloat16), w_ref[...],
              preferred_element_type=jnp.float32)
  hs_ref[...] = y[:, :D] + b_ref[...]
  hnA_ref[...] = y[:, D:D + DH]
  hnB_ref[...] = y[:, D + DH:]


def _tc3_body(hs_ref, aA_ref, aB_ref, d_ref, o_ref):
  invd = 1.0 / jnp.maximum(d_ref[...], 1.0)
  agg = jnp.concatenate([aA_ref[...], aB_ref[...]], axis=1) * invd
  o_ref[...] = hs_ref[...] + agg


def _row_spec(cols):
  return pl.BlockSpec((BM, cols), lambda i: (i, 0))


_W_SPEC = pl.BlockSpec((D, 2 * D), lambda i: (0, 0))
_B_SPEC = pl.BlockSpec((1, D), lambda i: (0, 0))
_D_SPEC = pl.BlockSpec((BM, 1), lambda i: (i, 0))

_tc1 = pl.pallas_call(
    _tc1_body,
    grid=(GRID_M,),
    in_specs=[_row_spec(D), _W_SPEC, _B_SPEC],
    out_specs=[_row_spec(D), _row_spec(DH), _row_spec(DH)],
    out_shape=[
        jax.ShapeDtypeStruct((N, D), jnp.float32),
        jax.ShapeDtypeStruct((N, DH), jnp.float32),
        jax.ShapeDtypeStruct((N, DH), jnp.float32),
    ],
    compiler_params=pltpu.CompilerParams(
        dimension_semantics=("parallel",)),
)

_tc2 = pl.pallas_call(
    _tc2_body,
    grid=(GRID_M,),
    in_specs=[_row_spec(D), _row_spec(DH), _row_spec(DH), _D_SPEC,
              _W_SPEC, _B_SPEC],
    out_specs=[_row_spec(D), _row_spec(DH), _row_spec(DH)],
    out_shape=[
        jax.ShapeDtypeStruct((N, D), jnp.float32),
        jax.ShapeDtypeStruct((N, DH), jnp.float32),
        jax.ShapeDtypeStruct((N, DH), jnp.float32),
    ],
    compiler_params=pltpu.CompilerParams(
        dimension_semantics=("parallel",)),
)

_tc3 = pl.pallas_call(
    _tc3_body,
    grid=(GRID_M,),
    in_specs=[_row_spec(D), _row_spec(DH), _row_spec(DH), _D_SPEC],
    out_specs=_row_spec(D),
    out_shape=jax.ShapeDtypeStruct((N, D), jnp.float32),
    compiler_params=pltpu.CompilerParams(
        dimension_semantics=("parallel",)),
)


@jax.jit
def kernel(x, edge_index, W1_self, W1_neigh, b1, W2_self, W2_neigh, b2):
  W1 = jnp.concatenate([W1_self, W1_neigh], axis=1).astype(jnp.bfloat16)
  W2 = jnp.concatenate([W2_self, W2_neigh], axis=1).astype(jnp.bfloat16)

  src = edge_index[0]
  dst = edge_index[1]
  pad = EPAD - E
  packed = jnp.concatenate([
      (dst << 16) | src,
      jnp.full((pad,), DUMMY << 16, jnp.int32),
  ]).reshape(NS, KB, BLK)

  z2 = jnp.zeros((ACC_R // NS, DH), jnp.float32)
  z1 = jnp.zeros((DEG_R // NS,), jnp.float32)
  ones = jnp.ones((BLK,), jnp.float32)

  xs, xnA, xnB = _tc1(x, W1, b1.reshape(1, D))
  aggA, aggB, deg = _sc_layer_deg(xnA, xnB, packed, z2, z1, ones)
  d = deg.reshape(DEG_R, 1)

  hs, hnA, hnB = _tc2(xs, aggA, aggB, d, W2, b2.reshape(1, D))
  a2A, a2B, _ = _sc_layer(hnA, hnB, packed, z2, z1, ones)
  out = _tc3(hs, a2A, a2B, d)
  return out


# speedup vs baseline: 5.6266x; 1.1064x over previous
import functools

import jax
import jax.numpy as jnp
from jax import lax
from jax.experimental import pallas as pl
from jax.experimental.pallas import tpu as pltpu
from jax.experimental.pallas import tpu_sc as plsc

N = 10000
E = 160000
D = 256
DH = 128

NC = 2
NS = 16
BLK = 128
KB = -(-E // (NS * BLK))
EPAD = NS * KB * BLK
ACC_R = 10112
DEG_R = 10240
DUMMY = N + 8


def _sc_layer_body(with_deg, tabA, tabB, pk_h, z2, z1, ones_h,
                   aggA_o, aggB_o, deg_o, acc, dacc, pk_v, sidx, didx,
                   rows_v, ones_v, gsem, ssem, dsem):
  c = lax.axis_index("c")
  s = lax.axis_index("s")

  pltpu.sync_copy(pk_h.at[s], pk_v)

  rz = ACC_R // NS
  pltpu.sync_copy(z2, acc.at[pl.ds(s * rz, rz)])
  if with_deg:
    @pl.when(c == 1)
    def _():
      dz = DEG_R // NS
      pltpu.sync_copy(z1, dacc.at[pl.ds(s * dz, dz)])
      pltpu.sync_copy(ones_h, ones_v)
  plsc.subcore_barrier()

  def unpack_idx(jb, buf):
    row = pk_v.at[jb]
    for i in range(BLK // 16):
      p = row[pl.ds(i * 16, 16)]
      sidx[buf, pl.ds(i * 16, 16)] = p & 0xFFFF
      didx[buf, pl.ds(i * 16, 16)] = lax.shift_right_logical(p, 16)

  def edge_loop(tab, do_deg):
    H = BLK // 2

    def issue_gather(buf):
      pltpu.async_copy(tab.at[sidx.at[buf, pl.ds(0, H)]],
                       rows_v.at[buf, pl.ds(0, H)], gsem)
      pltpu.async_copy(tab.at[sidx.at[buf, pl.ds(H, H)]],
                       rows_v.at[buf, pl.ds(H, H)], gsem)

    def wait_gather(buf):
      pltpu.make_async_copy(tab.at[sidx.at[buf, pl.ds(0, H)]],
                            rows_v.at[buf, pl.ds(0, H)], gsem).wait()
      pltpu.make_async_copy(tab.at[sidx.at[buf, pl.ds(H, H)]],
                            rows_v.at[buf, pl.ds(H, H)], gsem).wait()

    unpack_idx(0, 0)
    issue_gather(0)

    def step(j, carry):
      buf = lax.rem(j, 2)
      obuf = 1 - buf

      @pl.when(j >= 1)
      def _():
        pltpu.make_async_copy(rows_v.at[obuf], acc.at[didx.at[obuf]],
                              ssem).wait()
        if do_deg:
          pltpu.make_async_copy(ones_v, dacc.at[didx.at[obuf]],
                                dsem).wait()

      @pl.when(j + 1 < KB)
      def _():
        unpack_idx(j + 1, obuf)
        issue_gather(obuf)

      wait_gather(buf)
      pltpu.async_copy(rows_v.at[buf], acc.at[didx.at[buf]], ssem, add=True)
      if do_deg:
        pltpu.async_copy(ones_v, dacc.at[didx.at[buf]], dsem, add=True)
      return carry

    lax.fori_loop(0, KB, step, 0)
    fbuf = (KB - 1) % 2
    pltpu.make_async_copy(rows_v.at[fbuf], acc.at[didx.at[fbuf]],
                          ssem).wait()
    if do_deg:
      pltpu.make_async_copy(ones_v, dacc.at[didx.at[fbuf]], dsem).wait()

  @pl.when(c == 0)
  def _():
    edge_loop(tabA, False)

  @pl.when(c == 1)
  def _():
    edge_loop(tabB, with_deg)

  plsc.subcore_barrier()

  r0 = s * (ACC_R // NS)
  nr = ACC_R // NS

  @pl.when(c == 0)
  def _():
    pltpu.sync_copy(acc.at[pl.ds(r0, nr)], aggA_o.at[pl.ds(r0, nr)])

  @pl.when(c == 1)
  def _():
    pltpu.sync_copy(acc.at[pl.ds(r0, nr)], aggB_o.at[pl.ds(r0, nr)])
    if with_deg:
      d0 = s * (DEG_R // NS)
      pltpu.sync_copy(dacc.at[pl.ds(d0, DEG_R // NS)],
                      deg_o.at[pl.ds(d0, DEG_R // NS)])


def _make_sc_layer(with_deg):
  mesh = plsc.VectorSubcoreMesh(core_axis_name="c", subcore_axis_name="s",
                                num_cores=NC, num_subcores=NS)
  return pl.kernel(
      functools.partial(_sc_layer_body, with_deg),
      out_type=(
          jax.ShapeDtypeStruct((ACC_R, DH), jnp.float32),
          jax.ShapeDtypeStruct((ACC_R, DH), jnp.float32),
          jax.ShapeDtypeStruct((DEG_R,), jnp.float32),
      ),
      mesh=mesh,
      scratch_types=[
          pltpu.VMEM_SHARED((ACC_R, DH), jnp.float32),
          pltpu.VMEM_SHARED((DEG_R,), jnp.float32),
          pltpu.VMEM((KB, BLK), jnp.int32),
          pltpu.VMEM((2, BLK), jnp.int32),
          pltpu.VMEM((2, BLK), jnp.int32),
          pltpu.VMEM((2, BLK, DH), jnp.float32),
          pltpu.VMEM((BLK,), jnp.float32),
          pltpu.SemaphoreType.DMA,
          pltpu.SemaphoreType.DMA,
          pltpu.SemaphoreType.DMA,
      ],
      name="sage_segment_sum" + ("_deg" if with_deg else ""),
  )


_sc_layer_deg = _make_sc_layer(True)
_sc_layer = _make_sc_layer(False)


BM = 512
GRID_M = -(-N // BM)


def _tc1_body(x_ref, w_ref, b_ref, xs_ref, xnA_ref, xnB_ref):
  y = jnp.dot(x_ref[...].astype(jnp.bfloat16), w_ref[...],
              preferred_element_type=jnp.float32)
  xs_ref[...] = y[:, :D] + b_ref[...]
  xnA_ref[...] = y[:, D:D + DH]
  xnB_ref[...] = y[:, D + DH:]


def _tc2_body(xs_ref, aA_ref, aB_ref, d_ref, w_ref, b_ref,
              hs_ref, hnA_ref, hnB_ref):
  invd = 1.0 / jnp.maximum(d_ref[...], 1.0)
  agg = jnp.concatenate([aA_ref[...], aB_ref[...]], axis=1) * invd
  h = jnp.maximum(xs_ref[...] + agg, 0.0)
  y = jnp.dot(h.astype(jnp.bfloat16), w_ref[...],
              preferred_element_type=jnp.float32)
  hs_ref[...] = y[:, :D] + b_ref[...]
  hnA_ref[...] = y[:, D:D + DH]
  hnB_ref[...] = y[:, D + DH:]


def _tc3_body(hs_ref, aA_ref, aB_ref, d_ref, o_ref):
  invd = 1.0 / jnp.maximum(d_ref[...], 1.0)
  agg = jnp.concatenate([aA_ref[...], aB_ref[...]], axis=1) * invd
  o_ref[...] = hs_ref[...] + agg


def _row_spec(cols):
  return pl.BlockSpec((BM, cols), lambda i: (i, 0))


_W_SPEC = pl.BlockSpec((D, 2 * D), lambda i: (0, 0))
_B_SPEC = pl.BlockSpec((1, D), lambda i: (0, 0))
_D_SPEC = pl.BlockSpec((BM, 1), lambda i: (i, 0))

_tc1 = pl.pallas_call(
    _tc1_body,
    grid=(GRID_M,),
    in_specs=[_row_spec(D), _W_SPEC, _B_SPEC],
    out_specs=[_row_spec(D), _row_spec(DH), _row_spec(DH)],
    out_shape=[
        jax.ShapeDtypeStruct((N, D), jnp.float32),
        jax.ShapeDtypeStruct((N, DH), jnp.float32),
        jax.ShapeDtypeStruct((N, DH), jnp.float32),
    ],
    compiler_params=pltpu.CompilerParams(
        dimension_semantics=("parallel",)),
)

_tc2 = pl.pallas_call(
    _tc2_body,
    grid=(GRID_M,),
    in_specs=[_row_spec(D), _row_spec(DH), _row_spec(DH), _D_SPEC,
              _W_SPEC, _B_SPEC],
    out_specs=[_row_spec(D), _row_spec(DH), _row_spec(DH)],
    out_shape=[
        jax.ShapeDtypeStruct((N, D), jnp.float32),
        jax.ShapeDtypeStruct((N, DH), jnp.float32),
        jax.ShapeDtypeStruct((N, DH), jnp.float32),
    ],
    compiler_params=pltpu.CompilerParams(
        dimension_semantics=("parallel",)),
)

_tc3 = pl.pallas_call(
    _tc3_body,
    grid=(GRID_M,),
    in_specs=[_row_spec(D), _row_spec(DH), _row_spec(DH), _D_SPEC],
    out_specs=_row_spec(D),
    out_shape=jax.ShapeDtypeStruct((N, D), jnp.float32),
    compiler_params=pltpu.CompilerParams(
        dimension_semantics=("parallel",)),
)


@jax.jit
def kernel(x, edge_index, W1_self, W1_neigh, b1, W2_self, W2_neigh, b2):
  W1 = jnp.concatenate([W1_self, W1_neigh], axis=1).astype(jnp.bfloat16)
  W2 = jnp.concatenate([W2_self, W2_neigh], axis=1).astype(jnp.bfloat16)

  src = edge_index[0]
  dst = edge_index[1]
  pad = EPAD - E
  packed = jnp.concatenate([
      (dst << 16) | src,
      jnp.full((pad,), DUMMY << 16, jnp.int32),
  ]).reshape(NS, KB, BLK)

  z2 = jnp.zeros((ACC_R // NS, DH), jnp.float32)
  z1 = jnp.zeros((DEG_R // NS,), jnp.float32)
  ones = jnp.ones((BLK,), jnp.float32)

  xs, xnA, xnB = _tc1(x, W1, b1.reshape(1, D))
  aggA, aggB, deg = _sc_layer_deg(xnA, xnB, packed, z2, z1, ones)
  d = deg.reshape(DEG_R, 1)

  hs, hnA, hnB = _tc2(xs, aggA, aggB, d, W2, b2.reshape(1, D))
  a2A, a2B, _ = _sc_layer(hnA, hnB, packed, z2, z1, ones)
  out = _tc3(hs, a2A, a2B, d)
  return out
